# initial kernel scaffold (unmeasured)
import jax
import jax.numpy as jnp
from jax import lax
from jax.experimental import pallas as pl
from jax.experimental.pallas import tpu as pltpu


def kernel(
    x,
):
    def body(*refs):
        pass

    out_shape = jax.ShapeDtypeStruct(..., jnp.float32)
    return pl.pallas_call(body, out_shape=out_shape)(...)


# baseline (device time: 598916 ns/iter reference)
import jax
import jax.numpy as jnp
from jax import lax
from jax.experimental import pallas as pl
from jax.experimental.pallas import tpu as pltpu

N_DEV = 4


def kernel(x):
    x = x.reshape(x.shape[-2], x.shape[-1])
    m, n = x.shape
    ch = m // N_DEV
    n_steps = N_DEV - 1

    def body(x_ref, out_ref, comm_ref, send_sems, recv_sems, copy_sem):
        my = lax.axis_index("i")
        left = lax.rem(my + N_DEV - 1, N_DEV)
        right = lax.rem(my + 1, N_DEV)

        load = pltpu.make_async_copy(x_ref, out_ref, copy_sem)
        load.start()

        barrier_sem = pltpu.get_barrier_semaphore()
        for nbr in (left, right):
            pl.semaphore_signal(
                barrier_sem, inc=1,
                device_id=(nbr,), device_id_type=pl.DeviceIdType.MESH,
            )
        pl.semaphore_wait(barrier_sem, 2)
        load.wait()

        for s in range(n_steps):
            send_chunk = lax.rem(my + 2 * N_DEV - s, N_DEV)
            recv_chunk = lax.rem(my + 2 * N_DEV - s - 1, N_DEV)
            rdma = pltpu.make_async_remote_copy(
                src_ref=out_ref.at[pl.ds(send_chunk * ch, ch), :],
                dst_ref=comm_ref.at[s],
                send_sem=send_sems.at[s],
                recv_sem=recv_sems.at[s],
                device_id=(right,),
                device_id_type=pl.DeviceIdType.MESH,
            )
            rdma.start()
            rdma.wait()
            out_ref[pl.ds(recv_chunk * ch, ch), :] += comm_ref[s]

        for s in range(n_steps):
            send_chunk = lax.rem(my + 2 * N_DEV + 1 - s, N_DEV)
            sem = n_steps + s
            rdma = pltpu.make_async_remote_copy(
                src_ref=out_ref.at[pl.ds(send_chunk * ch, ch), :],
                dst_ref=out_ref.at[pl.ds(send_chunk * ch, ch), :],
                send_sem=send_sems.at[sem],
                recv_sem=recv_sems.at[sem],
                device_id=(right,),
                device_id_type=pl.DeviceIdType.MESH,
            )
            rdma.start()
            rdma.wait()

    return pl.pallas_call(
        body,
        out_shape=jax.ShapeDtypeStruct((m, n), x.dtype),
        in_specs=[pl.BlockSpec(memory_space=pl.ANY)],
        out_specs=pl.BlockSpec(memory_space=pltpu.VMEM),
        scratch_shapes=[
            pltpu.VMEM((N_DEV - 1, ch, n), x.dtype),
            pltpu.SemaphoreType.DMA((2 * n_steps,)),
            pltpu.SemaphoreType.DMA((2 * n_steps,)),
            pltpu.SemaphoreType.DMA,
        ],
        compiler_params=pltpu.CompilerParams(
            collective_id=0,
            vmem_limit_bytes=60 * 1024 * 1024,
        ),
    )(x)


# device time: 329535 ns/iter; 1.8175x vs baseline; 1.8175x over previous
import jax
import jax.numpy as jnp
from jax import lax
from jax.experimental import pallas as pl
from jax.experimental.pallas import tpu as pltpu

N_DEV = 4


def kernel(x):
    x = x.reshape(x.shape[-2], x.shape[-1])
    m, n = x.shape
    half = m // 2
    ch = half // N_DEV
    n_steps = N_DEV - 1

    def body(x_ref, out_ref, cw_ref, ccw_ref, send_sems, recv_sems, copy_sem):
        my = lax.axis_index("i")
        left = lax.rem(my + N_DEV - 1, N_DEV)
        right = lax.rem(my + 1, N_DEV)

        load = pltpu.make_async_copy(x_ref, out_ref, copy_sem)
        load.start()

        barrier_sem = pltpu.get_barrier_semaphore()
        for nbr in (left, right):
            pl.semaphore_signal(
                barrier_sem, inc=1,
                device_id=(nbr,), device_id_type=pl.DeviceIdType.MESH,
            )
        pl.semaphore_wait(barrier_sem, 2)
        load.wait()

        def cw_row(c):
            return c * ch

        def ccw_row(c):
            return half + c * ch

        for s in range(n_steps):
            cw_send = lax.rem(my + 2 * N_DEV - s, N_DEV)
            cw_recv = lax.rem(my + 2 * N_DEV - s - 1, N_DEV)
            ccw_send = lax.rem(my + s, N_DEV)
            ccw_recv = lax.rem(my + s + 1, N_DEV)
            cw = pltpu.make_async_remote_copy(
                src_ref=out_ref.at[pl.ds(cw_row(cw_send), ch), :],
                dst_ref=cw_ref.at[s],
                send_sem=send_sems.at[s],
                recv_sem=recv_sems.at[s],
                device_id=(right,),
                device_id_type=pl.DeviceIdType.MESH,
            )
            ccw = pltpu.make_async_remote_copy(
                src_ref=out_ref.at[pl.ds(ccw_row(ccw_send), ch), :],
                dst_ref=ccw_ref.at[s],
                send_sem=send_sems.at[2 * n_steps + s],
                recv_sem=recv_sems.at[2 * n_steps + s],
                device_id=(left,),
                device_id_type=pl.DeviceIdType.MESH,
            )
            cw.start()
            ccw.start()
            cw.wait()
            out_ref[pl.ds(cw_row(cw_recv), ch), :] += cw_ref[s]
            ccw.wait()
            out_ref[pl.ds(ccw_row(ccw_recv), ch), :] += ccw_ref[s]

        for s in range(n_steps):
            cw_send = lax.rem(my + 2 * N_DEV + 1 - s, N_DEV)
            ccw_send = lax.rem(my + 2 * N_DEV - 1 + s, N_DEV)
            cw = pltpu.make_async_remote_copy(
                src_ref=out_ref.at[pl.ds(cw_row(cw_send), ch), :],
                dst_ref=out_ref.at[pl.ds(cw_row(cw_send), ch), :],
                send_sem=send_sems.at[n_steps + s],
                recv_sem=recv_sems.at[n_steps + s],
                device_id=(right,),
                device_id_type=pl.DeviceIdType.MESH,
            )
            ccw = pltpu.make_async_remote_copy(
                src_ref=out_ref.at[pl.ds(ccw_row(ccw_send), ch), :],
                dst_ref=out_ref.at[pl.ds(ccw_row(ccw_send), ch), :],
                send_sem=send_sems.at[3 * n_steps + s],
                recv_sem=recv_sems.at[3 * n_steps + s],
                device_id=(left,),
                device_id_type=pl.DeviceIdType.MESH,
            )
            cw.start()
            ccw.start()
            cw.wait()
            ccw.wait()

    return pl.pallas_call(
        body,
        out_shape=jax.ShapeDtypeStruct((m, n), x.dtype),
        in_specs=[pl.BlockSpec(memory_space=pl.ANY)],
        out_specs=pl.BlockSpec(memory_space=pltpu.VMEM),
        scratch_shapes=[
            pltpu.VMEM((n_steps, ch, n), x.dtype),
            pltpu.VMEM((n_steps, ch, n), x.dtype),
            pltpu.SemaphoreType.DMA((4 * n_steps,)),
            pltpu.SemaphoreType.DMA((4 * n_steps,)),
            pltpu.SemaphoreType.DMA,
        ],
        compiler_params=pltpu.CompilerParams(
            collective_id=0,
            vmem_limit_bytes=60 * 1024 * 1024,
        ),
    )(x)


# device time: 318427 ns/iter; 1.8809x vs baseline; 1.0349x over previous
import jax
import jax.numpy as jnp
from jax import lax
from jax.experimental import pallas as pl
from jax.experimental.pallas import tpu as pltpu

N_DEV = 4
N_SUB = 2
N_HOPS = 2 * (N_DEV - 1)
N_RS = N_DEV - 1


def kernel(x):
    x = x.reshape(x.shape[-2], x.shape[-1])
    m, n = x.shape
    half = m // 2
    ch = half // N_DEV
    sch = ch // N_SUB

    def body(x_ref, out_ref, cw_ref, ccw_ref, send_sems, recv_sems, copy_sem):
        my = lax.axis_index("i")
        left = lax.rem(my + N_DEV - 1, N_DEV)
        right = lax.rem(my + 1, N_DEV)

        load = pltpu.make_async_copy(x_ref, out_ref, copy_sem)
        load.start()

        barrier_sem = pltpu.get_barrier_semaphore()
        for nbr in (left, right):
            pl.semaphore_signal(
                barrier_sem, inc=1,
                device_id=(nbr,), device_id_type=pl.DeviceIdType.MESH,
            )
        pl.semaphore_wait(barrier_sem, 2)
        load.wait()

        def cw_send_chunk(h):
            if h < N_RS:
                return lax.rem(my + 2 * N_DEV - h, N_DEV)
            return lax.rem(my + 2 * N_DEV + N_DEV - h, N_DEV)

        def ccw_send_chunk(h):
            if h < N_RS:
                return lax.rem(my + h, N_DEV)
            return lax.rem(my + 2 * N_DEV + h - N_DEV, N_DEV)

        def sem_idx(h, u, d):
            return (h * N_SUB + u) * 2 + d

        def mk_send(h, u, d):
            if d == 0:
                row0 = cw_send_chunk(h) * ch
                dev, land = right, cw_ref
            else:
                row0 = half + ccw_send_chunk(h) * ch
                dev, land = left, ccw_ref
            src = out_ref.at[pl.ds(row0 + u * sch, sch), :]
            if h < N_RS:
                dst = land.at[h, pl.ds(u * sch, sch), :]
            else:
                dst = src
            return pltpu.make_async_remote_copy(
                src_ref=src,
                dst_ref=dst,
                send_sem=send_sems.at[sem_idx(h, u, d)],
                recv_sem=recv_sems.at[sem_idx(h, u, d)],
                device_id=(dev,),
                device_id_type=pl.DeviceIdType.MESH,
            )

        pend = {}
        for u in range(N_SUB):
            for d in (0, 1):
                r = mk_send(0, u, d)
                r.start()
                pend[(u, d)] = r

        for h in range(N_HOPS):
            for u in range(N_SUB):
                for d in (0, 1):
                    pend[(u, d)].wait()
                    if h < N_RS:
                        if d == 0:
                            c = lax.rem(my + 2 * N_DEV - h - 1, N_DEV)
                            r0 = c * ch + u * sch
                            out_ref[pl.ds(r0, sch), :] += (
                                cw_ref[h, pl.ds(u * sch, sch), :]
                            )
                        else:
                            c = lax.rem(my + h + 1, N_DEV)
                            r0 = half + c * ch + u * sch
                            out_ref[pl.ds(r0, sch), :] += (
                                ccw_ref[h, pl.ds(u * sch, sch), :]
                            )
                    if h < N_HOPS - 1:
                        nxt = mk_send(h + 1, u, d)
                        nxt.start()
                        pend[(u, d)] = nxt

    return pl.pallas_call(
        body,
        out_shape=jax.ShapeDtypeStruct((m, n), x.dtype),
        in_specs=[pl.BlockSpec(memory_space=pl.ANY)],
        out_specs=pl.BlockSpec(memory_space=pltpu.VMEM),
        scratch_shapes=[
            pltpu.VMEM((N_RS, ch, n), x.dtype),
            pltpu.VMEM((N_RS, ch, n), x.dtype),
            pltpu.SemaphoreType.DMA((N_HOPS * N_SUB * 2,)),
            pltpu.SemaphoreType.DMA((N_HOPS * N_SUB * 2,)),
            pltpu.SemaphoreType.DMA,
        ],
        compiler_params=pltpu.CompilerParams(
            collective_id=0,
            vmem_limit_bytes=60 * 1024 * 1024,
        ),
    )(x)


# device time: 311779 ns/iter; 1.9210x vs baseline; 1.0213x over previous
import jax
import jax.numpy as jnp
from jax import lax
from jax.experimental import pallas as pl
from jax.experimental.pallas import tpu as pltpu

N_DEV = 4
N_SUB = 4
N_RS = N_DEV - 1
N_HOPS = 2 * (N_DEV - 1)


def kernel(x):
    x = x.reshape(x.shape[-2], x.shape[-1])
    m, n = x.shape
    half = m // 2
    ch = half // N_DEV
    sch = ch // N_SUB

    def body(x_ref, out_ref, cw_ref, ccw_ref, send_sems, recv_sems, copy_sems):
        my = lax.axis_index("i")
        left = lax.rem(my + N_DEV - 1, N_DEV)
        right = lax.rem(my + 1, N_DEV)

        loads = []
        for k in range(N_DEV):
            tr = lax.rem(my + 2 * N_DEV - k, N_DEV) * ch
            br = half + lax.rem(my + k, N_DEV) * ch
            lt = pltpu.make_async_copy(
                x_ref.at[pl.ds(tr, ch), :], out_ref.at[pl.ds(tr, ch), :],
                copy_sems.at[0, k])
            lb = pltpu.make_async_copy(
                x_ref.at[pl.ds(br, ch), :], out_ref.at[pl.ds(br, ch), :],
                copy_sems.at[1, k])
            lt.start()
            lb.start()
            loads.append((lt, lb))

        barrier_sem = pltpu.get_barrier_semaphore()
        for nbr in (left, right):
            pl.semaphore_signal(
                barrier_sem, inc=1,
                device_id=(nbr,), device_id_type=pl.DeviceIdType.MESH,
            )
        pl.semaphore_wait(barrier_sem, 2)
        loads[0][0].wait()
        loads[0][1].wait()

        def cw_send_chunk(h):
            if h < N_RS:
                return lax.rem(my + 2 * N_DEV - h, N_DEV)
            return lax.rem(my + 2 * N_DEV + N_DEV - h, N_DEV)

        def ccw_send_chunk(h):
            if h < N_RS:
                return lax.rem(my + h, N_DEV)
            return lax.rem(my + 2 * N_DEV + h - N_DEV, N_DEV)

        def sem_idx(h, u, d):
            return (h * N_SUB + u) * 2 + d

        def mk_send(h, u, d):
            if d == 0:
                row0 = cw_send_chunk(h) * ch
                dev, land = right, cw_ref
            else:
                row0 = half + ccw_send_chunk(h) * ch
                dev, land = left, ccw_ref
            src = out_ref.at[pl.ds(row0 + u * sch, sch), :]
            if h < N_RS:
                dst = land.at[h, pl.ds(u * sch, sch), :]
            else:
                dst = src
            return pltpu.make_async_remote_copy(
                src_ref=src,
                dst_ref=dst,
                send_sem=send_sems.at[sem_idx(h, u, d)],
                recv_sem=recv_sems.at[sem_idx(h, u, d)],
                device_id=(dev,),
                device_id_type=pl.DeviceIdType.MESH,
            )

        pend = {}
        for u in range(N_SUB):
            for d in (0, 1):
                r = mk_send(0, u, d)
                r.start()
                pend[(u, d)] = r

        for h in range(N_HOPS):
            if h < N_RS:
                loads[h + 1][0].wait()
                loads[h + 1][1].wait()
            for u in range(N_SUB):
                for d in (0, 1):
                    pend[(u, d)].wait()
                    if h < N_RS:
                        if d == 0:
                            c = lax.rem(my + 2 * N_DEV - h - 1, N_DEV)
                            r0 = c * ch + u * sch
                            out_ref[pl.ds(r0, sch), :] += (
                                cw_ref[h, pl.ds(u * sch, sch), :]
                            )
                        else:
                            c = lax.rem(my + h + 1, N_DEV)
                            r0 = half + c * ch + u * sch
                            out_ref[pl.ds(r0, sch), :] += (
                                ccw_ref[h, pl.ds(u * sch, sch), :]
                            )
                    if h < N_HOPS - 1:
                        nxt = mk_send(h + 1, u, d)
                        nxt.start()
                        pend[(u, d)] = nxt

    return pl.pallas_call(
        body,
        out_shape=jax.ShapeDtypeStruct((m, n), x.dtype),
        in_specs=[pl.BlockSpec(memory_space=pl.ANY)],
        out_specs=pl.BlockSpec(memory_space=pltpu.VMEM),
        scratch_shapes=[
            pltpu.VMEM((N_RS, ch, n), x.dtype),
            pltpu.VMEM((N_RS, ch, n), x.dtype),
            pltpu.SemaphoreType.DMA((N_HOPS * N_SUB * 2,)),
            pltpu.SemaphoreType.DMA((N_HOPS * N_SUB * 2,)),
            pltpu.SemaphoreType.DMA((2, N_DEV)),
        ],
        compiler_params=pltpu.CompilerParams(
            collective_id=0,
            vmem_limit_bytes=60 * 1024 * 1024,
        ),
    )(x)


# device time: 310162 ns/iter; 1.9310x vs baseline; 1.0052x over previous
import jax
import jax.numpy as jnp
from jax import lax
from jax.experimental import pallas as pl
from jax.experimental.pallas import tpu as pltpu

N_DEV = 4
N_SUB = 4
N_RS = N_DEV - 1
N_HOPS = 2 * (N_DEV - 1)


def kernel(x):
    x = x.reshape(x.shape[-2], x.shape[-1])
    m, n = x.shape
    half = m // 2
    ch = half // N_DEV
    sch = ch // N_SUB

    def body(x_ref, out_ref, cw_ref, ccw_ref, send_sems, recv_sems, copy_sems):
        my = lax.axis_index("i")
        left = lax.rem(my + N_DEV - 1, N_DEV)
        right = lax.rem(my + 1, N_DEV)

        loads = [None]
        for k in range(1, N_DEV):
            tr = lax.rem(my + 2 * N_DEV - k, N_DEV) * ch
            br = half + lax.rem(my + k, N_DEV) * ch
            lt = pltpu.make_async_copy(
                x_ref.at[pl.ds(tr, ch), :], out_ref.at[pl.ds(tr, ch), :],
                copy_sems.at[0, k])
            lb = pltpu.make_async_copy(
                x_ref.at[pl.ds(br, ch), :], out_ref.at[pl.ds(br, ch), :],
                copy_sems.at[1, k])
            lt.start()
            lb.start()
            loads.append((lt, lb))

        barrier_sem = pltpu.get_barrier_semaphore()
        for nbr in (left, right):
            pl.semaphore_signal(
                barrier_sem, inc=1,
                device_id=(nbr,), device_id_type=pl.DeviceIdType.MESH,
            )
        pl.semaphore_wait(barrier_sem, 2)

        def cw_send_chunk(h):
            if h < N_RS:
                return lax.rem(my + 2 * N_DEV - h, N_DEV)
            return lax.rem(my + 2 * N_DEV + N_DEV - h, N_DEV)

        def ccw_send_chunk(h):
            if h < N_RS:
                return lax.rem(my + h, N_DEV)
            return lax.rem(my + 2 * N_DEV + h - N_DEV, N_DEV)

        def sem_idx(h, u, d):
            return (h * N_SUB + u) * 2 + d

        def mk_send(h, u, d):
            if d == 0:
                row0 = cw_send_chunk(h) * ch
                dev, land = right, cw_ref
            else:
                row0 = half + ccw_send_chunk(h) * ch
                dev, land = left, ccw_ref
            if h == 0:
                src = x_ref.at[pl.ds(row0 + u * sch, sch), :]
            else:
                src = out_ref.at[pl.ds(row0 + u * sch, sch), :]
            if h < N_RS:
                dst = land.at[h, pl.ds(u * sch, sch), :]
            else:
                dst = src
            return pltpu.make_async_remote_copy(
                src_ref=src,
                dst_ref=dst,
                send_sem=send_sems.at[sem_idx(h, u, d)],
                recv_sem=recv_sems.at[sem_idx(h, u, d)],
                device_id=(dev,),
                device_id_type=pl.DeviceIdType.MESH,
            )

        pend = {}
        for u in range(N_SUB):
            for d in (0, 1):
                r = mk_send(0, u, d)
                r.start()
                pend[(u, d)] = r

        for h in range(N_HOPS):
            if h < N_RS:
                loads[h + 1][0].wait()
                loads[h + 1][1].wait()
            for u in range(N_SUB):
                for d in (0, 1):
                    pend[(u, d)].wait()
                    if h < N_RS:
                        if d == 0:
                            c = lax.rem(my + 2 * N_DEV - h - 1, N_DEV)
                            r0 = c * ch + u * sch
                            out_ref[pl.ds(r0, sch), :] += (
                                cw_ref[h, pl.ds(u * sch, sch), :]
                            )
                        else:
                            c = lax.rem(my + h + 1, N_DEV)
                            r0 = half + c * ch + u * sch
                            out_ref[pl.ds(r0, sch), :] += (
                                ccw_ref[h, pl.ds(u * sch, sch), :]
                            )
                    if h < N_HOPS - 1:
                        nxt = mk_send(h + 1, u, d)
                        nxt.start()
                        pend[(u, d)] = nxt

    return pl.pallas_call(
        body,
        out_shape=jax.ShapeDtypeStruct((m, n), x.dtype),
        in_specs=[pl.BlockSpec(memory_space=pl.ANY)],
        out_specs=pl.BlockSpec(memory_space=pltpu.VMEM),
        scratch_shapes=[
            pltpu.VMEM((N_RS, ch, n), x.dtype),
            pltpu.VMEM((N_RS, ch, n), x.dtype),
            pltpu.SemaphoreType.DMA((N_HOPS * N_SUB * 2,)),
            pltpu.SemaphoreType.DMA((N_HOPS * N_SUB * 2,)),
            pltpu.SemaphoreType.DMA((2, N_DEV)),
        ],
        compiler_params=pltpu.CompilerParams(
            collective_id=0,
            vmem_limit_bytes=60 * 1024 * 1024,
        ),
    )(x)


# device time: 310123 ns/iter; 1.9312x vs baseline; 1.0001x over previous
import jax
import jax.numpy as jnp
from jax import lax
from jax.experimental import pallas as pl
from jax.experimental.pallas import tpu as pltpu

N_DEV = 4
N_SUB = 4
N_RS = N_DEV - 1
N_HOPS = 2 * (N_DEV - 1)


def kernel(x):
    _, m, n = x.shape
    half = m // 2
    ch = half // N_DEV
    sch = ch // N_SUB

    def body(x_ref, out_ref, cw_ref, ccw_ref, send_sems, recv_sems, copy_sems):
        my = lax.axis_index("i")
        left = lax.rem(my + N_DEV - 1, N_DEV)
        right = lax.rem(my + 1, N_DEV)

        loads = [None]
        for k in range(1, N_DEV):
            tr = lax.rem(my + 2 * N_DEV - k, N_DEV) * ch
            br = half + lax.rem(my + k, N_DEV) * ch
            lt = pltpu.make_async_copy(
                x_ref.at[0, pl.ds(tr, ch), :], out_ref.at[pl.ds(tr, ch), :],
                copy_sems.at[0, k])
            lb = pltpu.make_async_copy(
                x_ref.at[0, pl.ds(br, ch), :], out_ref.at[pl.ds(br, ch), :],
                copy_sems.at[1, k])
            lt.start()
            lb.start()
            loads.append((lt, lb))

        barrier_sem = pltpu.get_barrier_semaphore()
        for nbr in (left, right):
            pl.semaphore_signal(
                barrier_sem, inc=1,
                device_id=(nbr,), device_id_type=pl.DeviceIdType.MESH,
            )
        pl.semaphore_wait(barrier_sem, 2)

        def cw_send_chunk(h):
            if h < N_RS:
                return lax.rem(my + 2 * N_DEV - h, N_DEV)
            return lax.rem(my + 2 * N_DEV + N_DEV - h, N_DEV)

        def ccw_send_chunk(h):
            if h < N_RS:
                return lax.rem(my + h, N_DEV)
            return lax.rem(my + 2 * N_DEV + h - N_DEV, N_DEV)

        def sem_idx(h, u, d):
            return (h * N_SUB + u) * 2 + d

        def mk_send(h, u, d):
            if d == 0:
                row0 = cw_send_chunk(h) * ch
                dev, land = right, cw_ref
            else:
                row0 = half + ccw_send_chunk(h) * ch
                dev, land = left, ccw_ref
            if h == 0:
                src = x_ref.at[0, pl.ds(row0 + u * sch, sch), :]
            else:
                src = out_ref.at[pl.ds(row0 + u * sch, sch), :]
            if h < N_RS:
                dst = land.at[h, pl.ds(u * sch, sch), :]
            else:
                dst = src
            return pltpu.make_async_remote_copy(
                src_ref=src,
                dst_ref=dst,
                send_sem=send_sems.at[sem_idx(h, u, d)],
                recv_sem=recv_sems.at[sem_idx(h, u, d)],
                device_id=(dev,),
                device_id_type=pl.DeviceIdType.MESH,
            )

        pend = {}
        for u in range(N_SUB):
            for d in (0, 1):
                r = mk_send(0, u, d)
                r.start()
                pend[(u, d)] = r

        for h in range(N_HOPS):
            if h < N_RS:
                loads[h + 1][0].wait()
                loads[h + 1][1].wait()
            for u in range(N_SUB):
                for d in (0, 1):
                    pend[(u, d)].wait()
                    if h < N_RS:
                        if d == 0:
                            c = lax.rem(my + 2 * N_DEV - h - 1, N_DEV)
                            r0 = c * ch + u * sch
                            out_ref[pl.ds(r0, sch), :] += (
                                cw_ref[h, pl.ds(u * sch, sch), :]
                            )
                        else:
                            c = lax.rem(my + h + 1, N_DEV)
                            r0 = half + c * ch + u * sch
                            out_ref[pl.ds(r0, sch), :] += (
                                ccw_ref[h, pl.ds(u * sch, sch), :]
                            )
                    if h < N_HOPS - 1:
                        nxt = mk_send(h + 1, u, d)
                        nxt.start()
                        pend[(u, d)] = nxt

    return pl.pallas_call(
        body,
        out_shape=jax.ShapeDtypeStruct((m, n), x.dtype),
        in_specs=[pl.BlockSpec(memory_space=pl.ANY)],
        out_specs=pl.BlockSpec(memory_space=pltpu.VMEM),
        scratch_shapes=[
            pltpu.VMEM((N_RS, ch, n), x.dtype),
            pltpu.VMEM((N_RS, ch, n), x.dtype),
            pltpu.SemaphoreType.DMA((N_HOPS * N_SUB * 2,)),
            pltpu.SemaphoreType.DMA((N_HOPS * N_SUB * 2,)),
            pltpu.SemaphoreType.DMA((2, N_DEV)),
        ],
        compiler_params=pltpu.CompilerParams(
            collective_id=0,
            vmem_limit_bytes=60 * 1024 * 1024,
        ),
    )(x)


# device time: 299437 ns/iter; 2.0001x vs baseline; 1.0357x over previous
import jax
import jax.numpy as jnp
from jax import lax
from jax.experimental import pallas as pl
from jax.experimental.pallas import tpu as pltpu

N_DEV = 4
N_SUB = 4
N_RS = N_DEV - 1
N_HOPS = 2 * (N_DEV - 1)


def kernel(x):
    _, m, n = x.shape
    half = m // 2
    ch = half // N_DEV
    sch = ch // N_SUB

    def body(x_ref, out_ref, acc_ref, cw_ref, ccw_ref,
             send_sems, recv_sems, copy_sems, store_sems):
        my = lax.axis_index("i")
        left = lax.rem(my + N_DEV - 1, N_DEV)
        right = lax.rem(my + 1, N_DEV)

        loads = [None]
        for k in range(1, N_DEV):
            tr = lax.rem(my + 2 * N_DEV - k, N_DEV) * ch
            br = half + lax.rem(my + k, N_DEV) * ch
            lt = pltpu.make_async_copy(
                x_ref.at[0, pl.ds(tr, ch), :], acc_ref.at[pl.ds(tr, ch), :],
                copy_sems.at[0, k])
            lb = pltpu.make_async_copy(
                x_ref.at[0, pl.ds(br, ch), :], acc_ref.at[pl.ds(br, ch), :],
                copy_sems.at[1, k])
            lt.start()
            lb.start()
            loads.append((lt, lb))

        barrier_sem = pltpu.get_barrier_semaphore()
        for nbr in (left, right):
            pl.semaphore_signal(
                barrier_sem, inc=1,
                device_id=(nbr,), device_id_type=pl.DeviceIdType.MESH,
            )
        pl.semaphore_wait(barrier_sem, 2)

        def cw_send_chunk(h):
            if h < N_RS:
                return lax.rem(my + 2 * N_DEV - h, N_DEV)
            return lax.rem(my + 2 * N_DEV + N_DEV - h, N_DEV)

        def ccw_send_chunk(h):
            if h < N_RS:
                return lax.rem(my + h, N_DEV)
            return lax.rem(my + 2 * N_DEV + h - N_DEV, N_DEV)

        def sem_idx(h, u, d):
            return (h * N_SUB + u) * 2 + d

        def mk_send(h, u, d):
            if d == 0:
                row0 = cw_send_chunk(h) * ch
                dev, land = right, cw_ref
            else:
                row0 = half + ccw_send_chunk(h) * ch
                dev, land = left, ccw_ref
            rows = pl.ds(row0 + u * sch, sch)
            if h == 0:
                src = x_ref.at[0, rows, :]
            elif h <= N_RS:
                src = acc_ref.at[rows, :]
            else:
                src = out_ref.at[rows, :]
            if h < N_RS:
                dst = land.at[h, pl.ds(u * sch, sch), :]
            else:
                dst = out_ref.at[rows, :]
            return pltpu.make_async_remote_copy(
                src_ref=src,
                dst_ref=dst,
                send_sem=send_sems.at[sem_idx(h, u, d)],
                recv_sem=recv_sems.at[sem_idx(h, u, d)],
                device_id=(dev,),
                device_id_type=pl.DeviceIdType.MESH,
            )

        pend = {}
        for u in range(N_SUB):
            for d in (0, 1):
                r = mk_send(0, u, d)
                r.start()
                pend[(u, d)] = r

        stores = []
        for h in range(N_HOPS):
            if h < N_RS:
                loads[h + 1][0].wait()
                loads[h + 1][1].wait()
            for u in range(N_SUB):
                for d in (0, 1):
                    pend[(u, d)].wait()
                    if h < N_RS:
                        if d == 0:
                            c = lax.rem(my + 2 * N_DEV - h - 1, N_DEV)
                            r0 = c * ch + u * sch
                            acc_ref[pl.ds(r0, sch), :] += (
                                cw_ref[h, pl.ds(u * sch, sch), :]
                            )
                        else:
                            c = lax.rem(my + h + 1, N_DEV)
                            r0 = half + c * ch + u * sch
                            acc_ref[pl.ds(r0, sch), :] += (
                                ccw_ref[h, pl.ds(u * sch, sch), :]
                            )
                    if h < N_HOPS - 1:
                        nxt = mk_send(h + 1, u, d)
                        nxt.start()
                        pend[(u, d)] = nxt
                    if h == N_RS - 1 and u == N_SUB - 1:
                        r0 = (lax.rem(my + 1, N_DEV) * ch if d == 0
                              else half + lax.rem(my + N_DEV - 1, N_DEV) * ch)
                        st = pltpu.make_async_copy(
                            acc_ref.at[pl.ds(r0, ch), :],
                            out_ref.at[pl.ds(r0, ch), :],
                            store_sems.at[d])
                        st.start()
                        stores.append(st)

        for st in stores:
            st.wait()

    return pl.pallas_call(
        body,
        out_shape=jax.ShapeDtypeStruct((m, n), x.dtype),
        in_specs=[pl.BlockSpec(memory_space=pl.ANY)],
        out_specs=pl.BlockSpec(memory_space=pl.ANY),
        scratch_shapes=[
            pltpu.VMEM((m, n), x.dtype),
            pltpu.VMEM((N_RS, ch, n), x.dtype),
            pltpu.VMEM((N_RS, ch, n), x.dtype),
            pltpu.SemaphoreType.DMA((N_HOPS * N_SUB * 2,)),
            pltpu.SemaphoreType.DMA((N_HOPS * N_SUB * 2,)),
            pltpu.SemaphoreType.DMA((2, N_DEV)),
            pltpu.SemaphoreType.DMA((2,)),
        ],
        compiler_params=pltpu.CompilerParams(
            collective_id=0,
            vmem_limit_bytes=60 * 1024 * 1024,
        ),
    )(x)
